# baseline (speedup 1.0000x reference)
import functools

import jax
import jax.numpy as jnp
from jax import lax
from jax.experimental import pallas as pl
from jax.experimental.pallas import tpu as pltpu
from jax.experimental.pallas import tpu_sc as plsc

_K = 4
_KP = 8
_TB = 512
_FP = 384
_NW = 32


def _rank_body(mag_ref, im_ref, choose_ref, carry_ref):
    nt = pl.program_id(1)

    @pl.when(nt == 0)
    def _init():
        carry_ref[...] = jnp.full(carry_ref.shape, 1e-8, jnp.float32)

    magb = mag_ref[0, 0]
    im = im_ref[0, 0]

    vals = jnp.dot(im.astype(jnp.bfloat16), magb.astype(jnp.bfloat16),
                   preferred_element_type=jnp.float32)
    n, tb = vals.shape
    idx2d = lax.broadcasted_iota(jnp.int32, (n, tb), 0)

    pos_rows = []
    v = vals
    for _ in range(_K):
        mx = jnp.max(v, axis=0)
        cand = jnp.where(v >= mx[None, :], idx2d, jnp.int32(2**30))
        ix = jnp.min(cand, axis=0)
        pos_rows.append(ix)
        v = jnp.where(idx2d == ix[None, :], jnp.float32(-1e30), v)
    posf = jnp.stack(pos_rows, axis=0).astype(jnp.float32)

    carry = carry_ref[0:_K, 0:2]
    pfull = jnp.concatenate([carry, posf], axis=1)
    carry_ref[0:_K, 0:2] = posf[:, tb - 2:tb]
    sm = (pfull[:, 0:tb] + pfull[:, 1:tb + 1] + pfull[:, 2:tb + 2]) / 3.0
    choose_ref[0, 0:_K, :] = sm.astype(jnp.int32)


def _rank_call(mag, integral_m):
    B, C, F, T = mag.shape
    N = integral_m.shape[2]
    return pl.pallas_call(
        _rank_body,
        grid=(B, T // _TB),
        in_specs=[
            pl.BlockSpec((1, 1, F, _TB), lambda b, t: (b, 0, 0, t)),
            pl.BlockSpec((1, 1, N, F), lambda b, t: (0, 0, 0, 0)),
        ],
        out_specs=pl.BlockSpec((1, _KP, _TB), lambda b, t: (b, 0, t)),
        out_shape=jax.ShapeDtypeStruct((B, _KP, T), jnp.int32),
        scratch_shapes=[pltpu.VMEM((8, 128), jnp.float32)],
        compiler_params=pltpu.CompilerParams(
            dimension_semantics=("arbitrary", "arbitrary"),
        ),
    )(mag, integral_m)


def _gather_body(table_hbm, choose_hbm, out_hbm, idx_v, rows_v, acc_v, sem):
    wid = lax.axis_index("s") * 2 + lax.axis_index("c")
    n_frames = out_hbm.shape[0]
    fpw = n_frames // _NW
    fbase = wid * fpw
    t_len = choose_hbm.shape[2]
    b = fbase // t_len
    t0 = fbase % t_len

    for k in range(_K):
        pltpu.sync_copy(choose_hbm.at[b, k, pl.ds(t0, fpw)], idx_v.at[k])

    nv = _FP // 16
    pltpu.async_copy(table_hbm.at[idx_v.at[0]], acc_v, sem).wait()
    for k in range(1, _K - 1):
        pltpu.async_copy(table_hbm.at[idx_v.at[k]], rows_v, sem).wait()

        def _add(j, _):
            for c in range(nv):
                s = pl.ds(c * 16, 16)
                acc_v[j, s] = acc_v[j, s] + rows_v[j, s]
            return 0

        lax.fori_loop(0, fpw, _add, 0)

    one = jnp.full((16,), 1.0, jnp.float32)
    zero = jnp.full((16,), 0.0, jnp.float32)
    pltpu.async_copy(table_hbm.at[idx_v.at[_K - 1]], rows_v, sem).wait()

    def _add_thr(j, _):
        for c in range(nv):
            s = pl.ds(c * 16, 16)
            a = acc_v[j, s] + rows_v[j, s]
            acc_v[j, s] = jnp.where(a > 0.0, one, zero)
        return 0

    lax.fori_loop(0, fpw, _add_thr, 0)
    pltpu.sync_copy(acc_v, out_hbm.at[pl.ds(fbase, fpw)])


def _gather_call(table_pad, choose):
    B, _, T = choose.shape
    fpw = (B * T) // _NW
    mesh = plsc.VectorSubcoreMesh(core_axis_name="c", subcore_axis_name="s")
    run = functools.partial(
        pl.kernel,
        mesh=mesh,
        out_type=jax.ShapeDtypeStruct((B * T, _FP), jnp.float32),
        scratch_types=[
            pltpu.VMEM((_K, fpw), jnp.int32),
            pltpu.VMEM((fpw, _FP), jnp.float32),
            pltpu.VMEM((fpw, _FP), jnp.float32),
            pltpu.SemaphoreType.DMA,
        ],
    )(_gather_body)
    return run(table_pad, choose)


@jax.jit
def _run(mag, integral_m, harmonic_loc):
    B, C, F, T = mag.shape
    table_pad = jnp.pad(harmonic_loc[0, 0], ((0, 0), (0, _FP - F)))
    outs = []
    for b in range(B):
        choose_b = _rank_call(mag[b:b + 1], integral_m)
        outs.append(_gather_call(table_pad, choose_b))
    out = jnp.stack(outs)[:, :, :F]
    return jnp.transpose(out, (0, 2, 1))[:, None]


def kernel(mag, integral_m, harmonic_loc, freq_dim):
    del freq_dim
    return _run(mag, integral_m, harmonic_loc)


# speedup vs baseline: 1.0380x; 1.0380x over previous
import functools

import jax
import jax.numpy as jnp
from jax import lax
from jax.experimental import pallas as pl
from jax.experimental.pallas import tpu as pltpu
from jax.experimental.pallas import tpu_sc as plsc

_K = 4
_KP = 8
_TB = 512
_FP = 384
_NW = 32


def _rank_body(mag_ref, im_ref, choose_ref, carry_ref):
    nt = pl.program_id(1)

    @pl.when(nt == 0)
    def _init():
        carry_ref[...] = jnp.full(carry_ref.shape, 1e-8, jnp.float32)

    magb = mag_ref[0, 0]
    im = im_ref[0, 0]

    vals = jnp.dot(im.astype(jnp.bfloat16), magb.astype(jnp.bfloat16),
                   preferred_element_type=jnp.float32)
    n, tb = vals.shape
    idx2d = lax.broadcasted_iota(jnp.int32, (n, tb), 0)

    pos_rows = []
    v = vals
    for _ in range(_K):
        mx = jnp.max(v, axis=0)
        cand = jnp.where(v >= mx[None, :], idx2d, jnp.int32(2**30))
        ix = jnp.min(cand, axis=0)
        pos_rows.append(ix)
        v = jnp.where(idx2d == ix[None, :], jnp.float32(-1e30), v)
    posf = jnp.stack(pos_rows, axis=0).astype(jnp.float32)

    carry = carry_ref[0:_K, 0:2]
    pfull = jnp.concatenate([carry, posf], axis=1)
    carry_ref[0:_K, 0:2] = posf[:, tb - 2:tb]
    sm = (pfull[:, 0:tb] + pfull[:, 1:tb + 1] + pfull[:, 2:tb + 2]) / 3.0
    choose_ref[0, 0:_K, :] = sm.astype(jnp.int32)


def _rank_call(mag, integral_m):
    B, C, F, T = mag.shape
    N = integral_m.shape[2]
    return pl.pallas_call(
        _rank_body,
        grid=(B, T // _TB),
        in_specs=[
            pl.BlockSpec((1, 1, F, _TB), lambda b, t: (b, 0, 0, t)),
            pl.BlockSpec((1, 1, N, F), lambda b, t: (0, 0, 0, 0)),
        ],
        out_specs=pl.BlockSpec((1, _KP, _TB), lambda b, t: (b, 0, t)),
        out_shape=jax.ShapeDtypeStruct((B, _KP, T), jnp.int32),
        scratch_shapes=[pltpu.VMEM((8, 128), jnp.float32)],
        compiler_params=pltpu.CompilerParams(
            dimension_semantics=("arbitrary", "arbitrary"),
        ),
    )(mag, integral_m)


def _gather_body(table_hbm, choose_hbm, out_hbm, idx_v, rows_v, acc_v, sem):
    wid = lax.axis_index("s") * 2 + lax.axis_index("c")
    n_frames = out_hbm.shape[0]
    fpw = n_frames // _NW
    fbase = wid * fpw
    t_len = choose_hbm.shape[2]
    b = fbase // t_len
    t0 = fbase % t_len

    for k in range(_K):
        pltpu.sync_copy(choose_hbm.at[b, k, pl.ds(t0, fpw)], idx_v.at[k])

    nv = _FP // 16
    pltpu.async_copy(table_hbm.at[idx_v.at[0]], acc_v, sem).wait()
    for k in range(1, _K - 1):
        pltpu.async_copy(table_hbm.at[idx_v.at[k]], rows_v, sem).wait()

        def _add(j, _):
            for c in range(nv):
                s = pl.ds(c * 16, 16)
                acc_v[j, s] = acc_v[j, s] + rows_v[j, s]
            return 0

        lax.fori_loop(0, fpw, _add, 0)

    one = jnp.full((16,), 1.0, jnp.float32)
    zero = jnp.full((16,), 0.0, jnp.float32)
    pltpu.async_copy(table_hbm.at[idx_v.at[_K - 1]], rows_v, sem).wait()

    def _add_thr(j, _):
        for c in range(nv):
            s = pl.ds(c * 16, 16)
            a = acc_v[j, s] + rows_v[j, s]
            acc_v[j, s] = jnp.where(a > 0.0, one, zero)
        return 0

    lax.fori_loop(0, fpw, _add_thr, 0)
    pltpu.sync_copy(acc_v, out_hbm.at[pl.ds(fbase, fpw)])


def _gather_call(table_pad, choose):
    B, _, T = choose.shape
    fpw = (B * T) // _NW
    mesh = plsc.VectorSubcoreMesh(core_axis_name="c", subcore_axis_name="s")
    run = functools.partial(
        pl.kernel,
        mesh=mesh,
        out_type=jax.ShapeDtypeStruct((B * T, _FP), jnp.float32),
        scratch_types=[
            pltpu.VMEM((_K, fpw), jnp.int32),
            pltpu.VMEM((fpw, _FP), jnp.float32),
            pltpu.VMEM((fpw, _FP), jnp.float32),
            pltpu.SemaphoreType.DMA,
        ],
    )(_gather_body)
    return run(table_pad, choose)


@jax.jit
def _run(mag, integral_m, harmonic_loc):
    B, C, F, T = mag.shape
    table_pad = jnp.pad(harmonic_loc[0, 0], ((0, 0), (0, _FP - F)))
    choose = _rank_call(mag, integral_m)
    out_t = _gather_call(table_pad, choose)
    out = out_t.reshape(B, T, _FP)[:, :, :F]
    return jnp.transpose(out, (0, 2, 1))[:, None]


def kernel(mag, integral_m, harmonic_loc, freq_dim):
    del freq_dim
    return _run(mag, integral_m, harmonic_loc)


# speedup vs baseline: 1.0625x; 1.0236x over previous
import functools

import jax
import jax.numpy as jnp
from jax import lax
from jax.experimental import pallas as pl
from jax.experimental.pallas import tpu as pltpu
from jax.experimental.pallas import tpu_sc as plsc

_K = 4
_KP = 8
_TB = 512
_FP = 384
_NW = 32


def _rank_body(mag_ref, im_ref, choose_ref, carry_ref):
    nt = pl.program_id(1)

    @pl.when(nt == 0)
    def _init():
        carry_ref[...] = jnp.full(carry_ref.shape, 1e-8, jnp.float32)

    magb = mag_ref[0, 0]
    im = im_ref[0, 0]

    vals = jnp.dot(im.astype(jnp.bfloat16), magb.astype(jnp.bfloat16),
                   preferred_element_type=jnp.float32)
    n, tb = vals.shape
    idx2d = lax.broadcasted_iota(jnp.int32, (n, tb), 0)

    pos_rows = []
    v = vals
    for _ in range(_K):
        mx = jnp.max(v, axis=0)
        cand = jnp.where(v >= mx[None, :], idx2d, jnp.int32(2**30))
        ix = jnp.min(cand, axis=0)
        pos_rows.append(ix)
        v = jnp.where(idx2d == ix[None, :], jnp.float32(-1e30), v)
    posf = jnp.stack(pos_rows, axis=0).astype(jnp.float32)

    carry = carry_ref[0:_K, 0:2]
    pfull = jnp.concatenate([carry, posf], axis=1)
    carry_ref[0:_K, 0:2] = posf[:, tb - 2:tb]
    sm = (pfull[:, 0:tb] + pfull[:, 1:tb + 1] + pfull[:, 2:tb + 2]) / 3.0
    choose_ref[0, 0:_K, :] = sm.astype(jnp.int32)


def _rank_call(mag, integral_m):
    B, C, F, T = mag.shape
    N = integral_m.shape[2]
    return pl.pallas_call(
        _rank_body,
        grid=(B, T // _TB),
        in_specs=[
            pl.BlockSpec((1, 1, F, _TB), lambda b, t: (b, 0, 0, t)),
            pl.BlockSpec((1, 1, N, F), lambda b, t: (0, 0, 0, 0)),
        ],
        out_specs=pl.BlockSpec((1, _KP, _TB), lambda b, t: (b, 0, t)),
        out_shape=jax.ShapeDtypeStruct((B, _KP, T), jnp.int32),
        scratch_shapes=[pltpu.VMEM((8, 128), jnp.float32)],
        compiler_params=pltpu.CompilerParams(
            dimension_semantics=("arbitrary", "arbitrary"),
        ),
    )(mag, integral_m)


_CH = 64


def _gather_body(table_hbm, choose_hbm, out_hbm, idx_v,
                 r0, r1, r2, r3, sem):
    wid = lax.axis_index("s") * 2 + lax.axis_index("c")
    n_frames = out_hbm.shape[0]
    fpw = n_frames // _NW
    fbase = wid * fpw
    t_len = choose_hbm.shape[2]
    b = fbase // t_len
    t0 = fbase % t_len

    for k in range(_K):
        pltpu.sync_copy(choose_hbm.at[b, k, pl.ds(t0, fpw)], idx_v.at[k])

    nv = _FP // 16
    one = jnp.full((16,), 1.0, jnp.float32)
    zero = jnp.full((16,), 0.0, jnp.float32)
    rows = (r0, r1, r2, r3)

    for ch in range(fpw // _CH):
        cps = [
            pltpu.async_copy(
                table_hbm.at[idx_v.at[k, pl.ds(ch * _CH, _CH)]],
                rows[k], sem)
            for k in range(_K)
        ]
        for cp in cps:
            cp.wait()

        def _comb(j, _):
            for c in range(nv):
                s = pl.ds(c * 16, 16)
                a = ((r0[j, s] + r1[j, s]) + (r2[j, s] + r3[j, s]))
                r0[j, s] = jnp.where(a > 0.0, one, zero)
            return 0

        lax.fori_loop(0, _CH, _comb, 0)
        pltpu.sync_copy(r0, out_hbm.at[pl.ds(fbase + ch * _CH, _CH)])


def _gather_call(table_pad, choose):
    B, _, T = choose.shape
    fpw = (B * T) // _NW
    mesh = plsc.VectorSubcoreMesh(core_axis_name="c", subcore_axis_name="s")
    run = functools.partial(
        pl.kernel,
        mesh=mesh,
        out_type=jax.ShapeDtypeStruct((B * T, _FP), jnp.float32),
        scratch_types=[
            pltpu.VMEM((_K, fpw), jnp.int32),
            pltpu.VMEM((_CH, _FP), jnp.float32),
            pltpu.VMEM((_CH, _FP), jnp.float32),
            pltpu.VMEM((_CH, _FP), jnp.float32),
            pltpu.VMEM((_CH, _FP), jnp.float32),
            pltpu.SemaphoreType.DMA,
        ],
    )(_gather_body)
    return run(table_pad, choose)


@jax.jit
def _run(mag, integral_m, harmonic_loc):
    B, C, F, T = mag.shape
    table_pad = jnp.pad(harmonic_loc[0, 0], ((0, 0), (0, _FP - F)))
    choose = _rank_call(mag, integral_m)
    out_t = _gather_call(table_pad, choose)
    out = out_t.reshape(B, T, _FP)[:, :, :F]
    return jnp.transpose(out, (0, 2, 1))[:, None]


def kernel(mag, integral_m, harmonic_loc, freq_dim):
    del freq_dim
    return _run(mag, integral_m, harmonic_loc)
